# baseline (speedup 1.0000x reference)
import functools

import jax
import jax.numpy as jnp
from jax import lax
from jax.experimental import pallas as pl
from jax.experimental.pallas import tpu as pltpu
from jax.experimental.pallas import tpu_sc as plsc


def _dist_argmax_body(xt_ref, et_ref, dist_ref, ind_ref):
    xbt = xt_ref[0]
    d = lax.dot_general(
        xbt, et_ref[...],
        dimension_numbers=(((0,), (0,)), ((), ())),
        preferred_element_type=jnp.float32,
    )
    dist_ref[...] = d.reshape(dist_ref.shape)
    ind_ref[...] = jnp.argmax(d, axis=1).astype(jnp.int32)


@functools.partial(jax.jit, static_argnames=("row_blk",))
def _dist_argmax(xt, embed_t, row_blk=256):
    b, d, n = xt.shape
    c = embed_t.shape[1]
    nblk = (b * n) // row_blk
    per_b = n // row_blk
    dist, ind = pl.pallas_call(
        _dist_argmax_body,
        grid=(nblk,),
        in_specs=[
            pl.BlockSpec((1, d, row_blk), lambda i: (i // per_b, 0, i % per_b)),
            pl.BlockSpec((d, c), lambda i: (0, 0)),
        ],
        out_specs=[
            pl.BlockSpec(
                (1, 1, row_blk, c), lambda i: (0, i // per_b, i % per_b, 0)
            ),
            pl.BlockSpec((row_blk,), lambda i: (i,)),
        ],
        out_shape=[
            jax.ShapeDtypeStruct((1, b, n, c), jnp.float32),
            jax.ShapeDtypeStruct((b * n,), jnp.int32),
        ],
    )(xt, embed_t)
    return dist, ind


def _make_sc_gather(v, d, bb, nn):
    nc, ns, lanes = 2, 16, 16
    nw = nc * ns
    b = bb * nn
    assert b % (8 * nw) == 0 and d % lanes == 0
    b_per_w = b // nw
    w_per_b = nn // b_per_w
    mesh = plsc.VectorSubcoreMesh(core_axis_name="c", subcore_axis_name="s")

    nchunk = 4
    cs = b_per_w // nchunk

    @functools.partial(
        pl.kernel,
        mesh=mesh,
        out_type=jax.ShapeDtypeStruct((bb, d, nn), jnp.float32),
        scratch_types=[
            pltpu.VMEM((b_per_w,), jnp.int32),
            pltpu.VMEM((b_per_w, d), jnp.float32),
            pltpu.VMEM((d, b_per_w), jnp.float32),
            pltpu.SemaphoreType.DMA,
            pltpu.SemaphoreType.DMA,
        ],
        compiler_params=pltpu.CompilerParams(
            use_tc_tiling_on_sc=False, needs_layout_passes=False
        ),
    )
    def gather(table_hbm, idx_hbm, out_hbm, idx_v, rows_v, tr_v, sem, osem):
        wid = lax.axis_index("s") * nc + lax.axis_index("c")
        base = wid * b_per_w
        pltpu.sync_copy(idx_hbm.at[pl.ds(base, b_per_w)], idx_v)

        gets = [
            pltpu.async_copy(
                table_hbm.at[idx_v.at[pl.ds(ck * cs, cs)]],
                rows_v.at[pl.ds(ck * cs, cs), :],
                sem,
            )
            for ck in range(nchunk)
        ]

        sh = lax.iota(jnp.int32, lanes)
        ob = wid // w_per_b
        ocol = (wid % w_per_b) * b_per_w
        puts = []
        for ck in range(nchunk):
            gets[ck].wait()

            def diag(j, _, ck=ck):
                cidx = lax.rem(j + sh, d)
                for k in range(cs // lanes):
                    ridx = ck * cs + k * lanes + sh
                    v = plsc.load_gather(rows_v, [ridx, cidx])
                    plsc.store_scatter(tr_v, [cidx, ridx], v)
                return 0

            lax.fori_loop(0, d, diag, 0)
            puts.append(
                pltpu.async_copy(
                    tr_v.at[:, pl.ds(ck * cs, cs)],
                    out_hbm.at[ob, :, pl.ds(ocol + ck * cs, cs)],
                    osem,
                )
            )
        for p in puts:
            p.wait()

    return gather


def kernel(x, embed):
    b, n, d = x.shape
    c = embed.shape[1]
    xt = jnp.transpose(x.astype(jnp.float32), (0, 2, 1))
    table = embed[0].astype(jnp.float32)

    dist, ind = _dist_argmax(xt, table.T)
    quantize_t = _make_sc_gather(c, d, b, n)(table, ind)

    return (
        jnp.transpose(quantize_t, (0, 2, 1)),
        ind.reshape(b, n),
        dist,
    )


# speedup vs baseline: 1.0021x; 1.0021x over previous
import functools

import jax
import jax.numpy as jnp
from jax import lax
from jax.experimental import pallas as pl
from jax.experimental.pallas import tpu as pltpu
from jax.experimental.pallas import tpu_sc as plsc


def _dist_argmax_body(xt_ref, et_ref, dist_ref, ind_ref):
    xbt = xt_ref[0]
    d = lax.dot_general(
        xbt, et_ref[...],
        dimension_numbers=(((0,), (0,)), ((), ())),
        preferred_element_type=jnp.float32,
    )
    dist_ref[...] = d.reshape(dist_ref.shape)
    ind_ref[...] = jnp.argmax(d, axis=1).astype(jnp.int32)


@functools.partial(jax.jit, static_argnames=("row_blk",))
def _dist_argmax(xt, embed_t, row_blk=256):
    b, d, n = xt.shape
    c = embed_t.shape[1]
    nblk = (b * n) // row_blk
    per_b = n // row_blk
    dist, ind = pl.pallas_call(
        _dist_argmax_body,
        grid=(nblk,),
        in_specs=[
            pl.BlockSpec((1, d, row_blk), lambda i: (i // per_b, 0, i % per_b)),
            pl.BlockSpec((d, c), lambda i: (0, 0)),
        ],
        out_specs=[
            pl.BlockSpec(
                (1, 1, row_blk, c), lambda i: (0, i // per_b, i % per_b, 0)
            ),
            pl.BlockSpec((row_blk,), lambda i: (i,)),
        ],
        out_shape=[
            jax.ShapeDtypeStruct((1, b, n, c), jnp.float32),
            jax.ShapeDtypeStruct((b * n,), jnp.int32),
        ],
    )(xt, embed_t)
    return dist, ind


def _make_sc_gather(v, d, bb, nn):
    nc, ns, lanes = 2, 16, 16
    nw = nc * ns
    b = bb * nn
    assert b % (8 * nw) == 0 and d % lanes == 0
    b_per_w = b // nw
    w_per_b = nn // b_per_w
    mesh = plsc.VectorSubcoreMesh(core_axis_name="c", subcore_axis_name="s")

    nchunk = 2
    cs = b_per_w // nchunk

    @functools.partial(
        pl.kernel,
        mesh=mesh,
        out_type=jax.ShapeDtypeStruct((bb, d, nn), jnp.float32),
        scratch_types=[
            pltpu.VMEM((b_per_w,), jnp.int32),
            pltpu.VMEM((b_per_w, d), jnp.float32),
            pltpu.VMEM((d, b_per_w), jnp.float32),
            pltpu.SemaphoreType.DMA,
            pltpu.SemaphoreType.DMA,
        ],
        compiler_params=pltpu.CompilerParams(
            use_tc_tiling_on_sc=False, needs_layout_passes=False
        ),
    )
    def gather(table_hbm, idx_hbm, out_hbm, idx_v, rows_v, tr_v, sem, osem):
        wid = lax.axis_index("s") * nc + lax.axis_index("c")
        base = wid * b_per_w
        pltpu.sync_copy(idx_hbm.at[pl.ds(base, b_per_w)], idx_v)

        gets = [
            pltpu.async_copy(
                table_hbm.at[idx_v.at[pl.ds(ck * cs, cs)]],
                rows_v.at[pl.ds(ck * cs, cs), :],
                sem,
            )
            for ck in range(nchunk)
        ]

        sh = lax.iota(jnp.int32, lanes)
        ob = wid // w_per_b
        ocol = (wid % w_per_b) * b_per_w
        puts = []
        for ck in range(nchunk):
            gets[ck].wait()

            def diag(j, _, ck=ck):
                cidx = lax.rem(j + sh, d)
                for k in range(cs // lanes):
                    ridx = ck * cs + k * lanes + sh
                    v = plsc.load_gather(rows_v, [ridx, cidx])
                    plsc.store_scatter(tr_v, [cidx, ridx], v)
                return 0

            lax.fori_loop(0, d, diag, 0)
            puts.append(
                pltpu.async_copy(
                    tr_v.at[:, pl.ds(ck * cs, cs)],
                    out_hbm.at[ob, :, pl.ds(ocol + ck * cs, cs)],
                    osem,
                )
            )
        for p in puts:
            p.wait()

    return gather


def kernel(x, embed):
    b, n, d = x.shape
    c = embed.shape[1]
    xt = jnp.transpose(x.astype(jnp.float32), (0, 2, 1))
    table = embed[0].astype(jnp.float32)

    dist, ind = _dist_argmax(xt, table.T)
    quantize_t = _make_sc_gather(c, d, b, n)(table, ind)

    return (
        jnp.transpose(quantize_t, (0, 2, 1)),
        ind.reshape(b, n),
        dist,
    )


# speedup vs baseline: 1.0334x; 1.0313x over previous
import functools

import jax
import jax.numpy as jnp
from jax import lax
from jax.experimental import pallas as pl
from jax.experimental.pallas import tpu as pltpu
from jax.experimental.pallas import tpu_sc as plsc


def _dist_argmax_body(xt_ref, et_ref, dist_ref, ind_ref):
    xbt = xt_ref[0]
    d = lax.dot_general(
        xbt, et_ref[...],
        dimension_numbers=(((0,), (0,)), ((), ())),
        preferred_element_type=jnp.float32,
    )
    dist_ref[...] = d.reshape(dist_ref.shape)
    ind_ref[...] = jnp.argmax(d, axis=1).astype(jnp.int32)


@functools.partial(jax.jit, static_argnames=("row_blk",))
def _dist_argmax(xt, embed_t, row_blk=256):
    b, d, n = xt.shape
    c = embed_t.shape[1]
    nblk = (b * n) // row_blk
    per_b = n // row_blk
    dist, ind = pl.pallas_call(
        _dist_argmax_body,
        grid=(nblk,),
        in_specs=[
            pl.BlockSpec((1, d, row_blk), lambda i: (i // per_b, 0, i % per_b)),
            pl.BlockSpec((d, c), lambda i: (0, 0)),
        ],
        out_specs=[
            pl.BlockSpec(
                (1, 1, row_blk, c), lambda i: (0, i // per_b, i % per_b, 0)
            ),
            pl.BlockSpec((row_blk,), lambda i: (i,)),
        ],
        out_shape=[
            jax.ShapeDtypeStruct((1, b, n, c), jnp.float32),
            jax.ShapeDtypeStruct((b * n,), jnp.int32),
        ],
    )(xt, embed_t)
    return dist, ind


def _make_sc_table(c, d):
    nc, ns, lanes = 2, 16, 16
    nw = nc * ns
    cols_per_w = c // nw
    mesh = plsc.VectorSubcoreMesh(core_axis_name="c", subcore_axis_name="s")

    @functools.partial(
        pl.kernel,
        mesh=mesh,
        out_type=jax.ShapeDtypeStruct((c * d,), jnp.float32),
        scratch_types=[
            pltpu.VMEM((d, cols_per_w), jnp.float32),
            pltpu.VMEM((cols_per_w * d,), jnp.float32),
        ],
        compiler_params=pltpu.CompilerParams(
            use_tc_tiling_on_sc=True, needs_layout_passes=False
        ),
    )
    def table_transpose(et_hbm, out_hbm, buf, tr_lin):
        wid = lax.axis_index("s") * nc + lax.axis_index("c")
        col0 = wid * cols_per_w
        pltpu.sync_copy(et_hbm.at[:, pl.ds(col0, cols_per_w)], buf)
        sh = lax.iota(jnp.int32, lanes)

        def diag(j, _):
            cidx = lax.rem(j + sh, d)
            for k in range(cols_per_w // lanes):
                ridx = k * lanes + sh
                v = plsc.load_gather(buf, [cidx, ridx])
                plsc.store_scatter(tr_lin, [ridx * d + cidx], v)
            return 0

        lax.fori_loop(0, d, diag, 0)
        pltpu.sync_copy(tr_lin, out_hbm.at[pl.ds(col0 * d, cols_per_w * d)])

    return table_transpose


def _make_sc_gather(v, d, bb, nn):
    nc, ns, lanes = 2, 16, 16
    nw = nc * ns
    b = bb * nn
    assert b % (8 * nw) == 0 and d % lanes == 0
    b_per_w = b // nw
    w_per_b = nn // b_per_w
    mesh = plsc.VectorSubcoreMesh(core_axis_name="c", subcore_axis_name="s")

    nchunk = 2
    cs = b_per_w // nchunk

    @functools.partial(
        pl.kernel,
        mesh=mesh,
        out_type=jax.ShapeDtypeStruct((bb, d, nn), jnp.float32),
        scratch_types=[
            pltpu.VMEM((b_per_w,), jnp.int32),
            pltpu.VMEM((b_per_w, d), jnp.float32),
            pltpu.VMEM((d, b_per_w), jnp.float32),
            pltpu.SemaphoreType.DMA,
            pltpu.SemaphoreType.DMA,
        ],
        compiler_params=pltpu.CompilerParams(
            use_tc_tiling_on_sc=False, needs_layout_passes=False
        ),
    )
    def gather(table_hbm, idx_hbm, out_hbm, idx_v, rows_v, tr_v, sem, osem):
        wid = lax.axis_index("s") * nc + lax.axis_index("c")
        base = wid * b_per_w
        pltpu.sync_copy(idx_hbm.at[pl.ds(base, b_per_w)], idx_v)

        gets = [
            pltpu.async_copy(
                table_hbm.at[idx_v.at[pl.ds(ck * cs, cs)]],
                rows_v.at[pl.ds(ck * cs, cs), :],
                sem,
            )
            for ck in range(nchunk)
        ]

        sh = lax.iota(jnp.int32, lanes)
        ob = wid // w_per_b
        ocol = (wid % w_per_b) * b_per_w
        puts = []
        for ck in range(nchunk):
            gets[ck].wait()

            def diag(j, _, ck=ck):
                cidx = lax.rem(j + sh, d)
                for k in range(cs // lanes):
                    ridx = ck * cs + k * lanes + sh
                    v = plsc.load_gather(rows_v, [ridx, cidx])
                    plsc.store_scatter(tr_v, [cidx, ridx], v)
                return 0

            lax.fori_loop(0, d, diag, 0)
            puts.append(
                pltpu.async_copy(
                    tr_v.at[:, pl.ds(ck * cs, cs)],
                    out_hbm.at[ob, :, pl.ds(ocol + ck * cs, cs)],
                    osem,
                )
            )
        for p in puts:
            p.wait()

    return gather


def kernel(x, embed):
    b, n, d = x.shape
    c = embed.shape[1]
    xt = jnp.transpose(x.astype(jnp.float32), (0, 2, 1))
    embed_t = embed[0].astype(jnp.float32).T

    tab_lin = _make_sc_table(c, d)(embed_t)
    dist, ind = _dist_argmax(xt, embed_t)
    table = tab_lin.reshape(c, d)
    quantize_t = _make_sc_gather(c, d, b, n)(table, ind)

    return (
        jnp.transpose(quantize_t, (0, 2, 1)),
        ind.reshape(b, n),
        dist,
    )


# speedup vs baseline: 1.0566x; 1.0224x over previous
import functools

import jax
import jax.numpy as jnp
from jax import lax
from jax.experimental import pallas as pl
from jax.experimental.pallas import tpu as pltpu
from jax.experimental.pallas import tpu_sc as plsc


def _dist_argmax_body(xt_ref, et_ref, dist_ref, ind_ref):
    xbt = xt_ref[0]
    d = lax.dot_general(
        xbt, et_ref[...],
        dimension_numbers=(((0,), (0,)), ((), ())),
        preferred_element_type=jnp.float32,
    )
    dist_ref[...] = d.reshape(dist_ref.shape)
    ind_ref[...] = jnp.argmax(d, axis=1).astype(jnp.int32)


@functools.partial(jax.jit, static_argnames=("row_blk",))
def _dist_argmax(xt, embed_t, row_blk=256):
    b, d, n = xt.shape
    c = embed_t.shape[1]
    nblk = (b * n) // row_blk
    per_b = n // row_blk
    dist, ind = pl.pallas_call(
        _dist_argmax_body,
        grid=(nblk,),
        in_specs=[
            pl.BlockSpec((1, d, row_blk), lambda i: (i // per_b, 0, i % per_b)),
            pl.BlockSpec((d, c), lambda i: (0, 0)),
        ],
        out_specs=[
            pl.BlockSpec(
                (1, 1, row_blk, c), lambda i: (0, i // per_b, i % per_b, 0)
            ),
            pl.BlockSpec((row_blk,), lambda i: (i,)),
        ],
        out_shape=[
            jax.ShapeDtypeStruct((1, b, n, c), jnp.float32),
            jax.ShapeDtypeStruct((b * n,), jnp.int32),
        ],
    )(xt, embed_t)
    return dist, ind


def _make_sc_table(c, d):
    nc, ns, lanes = 2, 16, 16
    nw = nc * ns
    cols_per_w = c // nw
    mesh = plsc.VectorSubcoreMesh(core_axis_name="c", subcore_axis_name="s")

    @functools.partial(
        pl.kernel,
        mesh=mesh,
        out_type=jax.ShapeDtypeStruct((c * d,), jnp.float32),
        scratch_types=[
            pltpu.VMEM((d, cols_per_w), jnp.float32),
            pltpu.VMEM((cols_per_w * d,), jnp.float32),
        ],
        compiler_params=pltpu.CompilerParams(
            use_tc_tiling_on_sc=True, needs_layout_passes=False
        ),
    )
    def table_transpose(et_hbm, out_hbm, buf, tr_lin):
        wid = lax.axis_index("s") * nc + lax.axis_index("c")
        col0 = wid * cols_per_w
        pltpu.sync_copy(et_hbm.at[:, pl.ds(col0, cols_per_w)], buf)
        sh = lax.iota(jnp.int32, lanes)

        def diag(j, _):
            cidx = lax.rem(j + sh, d)
            for k in range(cols_per_w // lanes):
                ridx = k * lanes + sh
                v = plsc.load_gather(buf, [cidx, ridx])
                plsc.store_scatter(tr_lin, [ridx * d + cidx], v)
            return 0

        lax.fori_loop(0, d, diag, 0)
        pltpu.sync_copy(tr_lin, out_hbm.at[pl.ds(col0 * d, cols_per_w * d)])

    return table_transpose


def _make_sc_gather(v, d, bb, nn):
    nc, ns, lanes = 2, 16, 16
    nw = nc * ns
    b = bb * nn
    assert b % (8 * nw) == 0 and d % lanes == 0
    b_per_w = b // nw
    w_per_b = nn // b_per_w
    mesh = plsc.VectorSubcoreMesh(core_axis_name="c", subcore_axis_name="s")

    nchunk = 2
    cs = b_per_w // nchunk
    st = d // 8

    @functools.partial(
        pl.kernel,
        mesh=mesh,
        out_type=jax.ShapeDtypeStruct((bb, st, nn // 128, 8, 128), jnp.float32),
        scratch_types=[
            pltpu.VMEM((b_per_w,), jnp.int32),
            pltpu.VMEM((b_per_w, d), jnp.float32),
            pltpu.VMEM((st, nchunk, 8, 128), jnp.float32),
            pltpu.SemaphoreType.DMA,
            pltpu.SemaphoreType.DMA,
        ],
        compiler_params=pltpu.CompilerParams(
            use_tc_tiling_on_sc=False, needs_layout_passes=False
        ),
    )
    def gather(table_hbm, idx_hbm, out_hbm, idx_v, rows_v, tile_v, sem, osem):
        wid = lax.axis_index("s") * nc + lax.axis_index("c")
        base = wid * b_per_w
        pltpu.sync_copy(idx_hbm.at[pl.ds(base, b_per_w)], idx_v)

        gets = [
            pltpu.async_copy(
                table_hbm.at[idx_v.at[pl.ds(ck * cs, cs)]],
                rows_v.at[pl.ds(ck * cs, cs), :],
                sem,
            )
            for ck in range(nchunk)
        ]

        sh = lax.iota(jnp.int32, lanes)
        ob = wid // w_per_b
        t0 = (wid % w_per_b) * b_per_w // 128
        puts = []
        for ck in range(nchunk):
            gets[ck].wait()

            def diag(j, _, ck=ck):
                cidx = lax.rem(j + sh, d)
                for k in range(cs // lanes):
                    ridx = ck * cs + k * lanes + sh
                    v = plsc.load_gather(rows_v, [ridx, cidx])
                    plsc.store_scatter(
                        tile_v,
                        [cidx // 8, jnp.full((lanes,), ck, jnp.int32),
                         cidx % 8, ridx % 128],
                        v,
                    )
                return 0

            lax.fori_loop(0, d, diag, 0)
            puts.append(
                pltpu.async_copy(
                    tile_v.at[:, pl.ds(ck, 1), :, :],
                    out_hbm.at[ob, :, pl.ds(t0 + ck, 1), :, :],
                    osem,
                )
            )
        for p in puts:
            p.wait()

    return gather


def kernel(x, embed):
    b, n, d = x.shape
    c = embed.shape[1]
    xt = jnp.transpose(x.astype(jnp.float32), (0, 2, 1))
    embed_t = embed[0].astype(jnp.float32).T

    tab_lin = _make_sc_table(c, d)(embed_t)
    dist, ind = _dist_argmax(xt, embed_t)
    table = tab_lin.reshape(c, d)
    qt = _make_sc_gather(c, d, b, n)(table, ind)

    return (
        jnp.transpose(qt, (0, 2, 4, 1, 3)).reshape(b, n, d),
        ind.reshape(b, n),
        dist,
    )
